# baseline (speedup 1.0000x reference)
import jax
import jax.numpy as jnp
from jax.experimental import pallas as pl

N = 50000
HID = 64
HEADS = 4
OUT_C = HID // HEADS
LAYERS = 3
LAT = 32


def _gat_layer(h, src, dst, e_h, p):
    xh = (h @ p["W"]).reshape(-1, HEADS, OUT_C)
    a_src = (xh * p["att_src"]).sum(-1)
    a_dst = (xh * p["att_dst"]).sum(-1)
    eh = (e_h @ p["W_e"]).reshape(-1, HEADS, OUT_C)
    a_e = (eh * p["att_e"]).sum(-1)
    alpha = a_src[src] + a_dst[dst] + a_e
    alpha = jax.nn.leaky_relu(alpha, 0.2)
    amax = jax.ops.segment_max(alpha, dst, num_segments=N)
    amax = jnp.where(jnp.isfinite(amax), amax, 0.0)
    ex = jnp.exp(alpha - amax[dst])
    denom = jax.ops.segment_sum(ex, dst, num_segments=N)
    w = ex / (denom[dst] + 1e-16)
    msg = xh[src] * w[..., None]
    out = jax.ops.segment_sum(msg, dst, num_segments=N)
    return out.reshape(-1, HEADS * OUT_C) + p["b"]


def _final_kernel(h_ref, muw_ref, mub_ref, lvw_ref, lvb_ref, mu_ref, lv_ref):
    hm = jnp.mean(h_ref[...], axis=0, keepdims=True)
    mu_ref[...] = hm @ muw_ref[...] + mub_ref[...][None, :]
    lv_ref[...] = hm @ lvw_ref[...] + lvb_ref[...][None, :]


def kernel(x, face_types, edge_index, edge_attr, params):
    fe = params["face_emb"][face_types]
    xc = jnp.concatenate([x, fe], axis=-1)
    h = jax.nn.relu(xc @ params["node_W"] + params["node_b"])
    e_h = jax.nn.relu(edge_attr @ params["edge_W"] + params["edge_b"])
    src, dst = edge_index[0], edge_index[1]
    for l in range(LAYERS):
        h = jax.nn.relu(_gat_layer(h, src, dst, e_h, params["gat"][l]))
    mu, lv = pl.pallas_call(
        _final_kernel,
        out_shape=(
            jax.ShapeDtypeStruct((1, LAT), jnp.float32),
            jax.ShapeDtypeStruct((1, LAT), jnp.float32),
        ),
    )(h, params["mu_W"], params["mu_b"], params["lv_W"], params["lv_b"])
    return (mu, lv)


# speedup vs baseline: 29.7933x; 29.7933x over previous
import functools

import jax
import jax.numpy as jnp
from jax import lax
from jax.experimental import pallas as pl
from jax.experimental.pallas import tpu as pltpu
from jax.experimental.pallas import tpu_sc as plsc

N = 50000
E = 800000
NODE_F = 13
EDGE_F = 2
FACE_D = 8
HID = 64
HEADS = 4
OUT_C = HID // HEADS
LAYERS = 3
LAT = 32

NC = 2
NS = 16
CH = 128

ROWS = 6272
E_PAD = ROWS * CH
ROWS_A = ROWS // (NC * NS)
ROWS_B = ROWS // NS
N_PAD = 50176
NPT = N_PAD // NS

RB = 1792
RBE = ROWS // 16


def _k0_body(x_ref, ft_ref, emb_ref, w_ref, b_ref, out_ref):
    xb = x_ref[...]
    ft = ft_ref[...]
    emb = emb_ref[...]
    fe = jnp.where(ft == 0, emb[0][None, :],
                   jnp.where(ft == 1, emb[1][None, :], emb[2][None, :]))
    xc = jnp.concatenate([xb, fe], axis=1)
    h = jnp.maximum(xc @ w_ref[...] + b_ref[...], 0.0)
    for k in range(HEADS):
        out_ref[k] = h[:, 16 * k:16 * k + 16]


def _node_init(x_p, ft_p, emb, w, b):
    return pl.pallas_call(
        _k0_body,
        grid=(N_PAD // RB,),
        in_specs=[
            pl.BlockSpec((RB, NODE_F), lambda i: (i, 0)),
            pl.BlockSpec((RB, 1), lambda i: (i, 0)),
            pl.BlockSpec((3, FACE_D), lambda i: (0, 0)),
            pl.BlockSpec((NODE_F + FACE_D, HID), lambda i: (0, 0)),
            pl.BlockSpec((1, HID), lambda i: (0, 0)),
        ],
        out_specs=pl.BlockSpec((HEADS, RB, 16), lambda i: (0, i, 0)),
        out_shape=jax.ShapeDtypeStruct((HEADS, N_PAD, 16), jnp.float32),
    )(x_p, ft_p, emb, w, b)


def _ke_body(ea_ref, wt_ref, b_ref, awt_ref, o0_ref, o1_ref, o2_ref):
    eh = jnp.maximum(wt_ref[...] @ ea_ref[...] + b_ref[...], 0.0)
    a = awt_ref[...] @ eh
    for l, o in enumerate((o0_ref, o1_ref, o2_ref)):
        o[...] = a[l * HEADS:(l + 1) * HEADS].reshape(HEADS, RBE, CH)


def _edge_logits(ea_t, wt, b, awt):
    return pl.pallas_call(
        _ke_body,
        grid=(ROWS // RBE,),
        in_specs=[
            pl.BlockSpec((EDGE_F, RBE * CH), lambda i: (0, i)),
            pl.BlockSpec((HID, EDGE_F), lambda i: (0, 0)),
            pl.BlockSpec((HID, 1), lambda i: (0, 0)),
            pl.BlockSpec((LAYERS * HEADS, HID), lambda i: (0, 0)),
        ],
        out_specs=[pl.BlockSpec((HEADS, RBE, CH), lambda i: (0, i, 0))] * 3,
        out_shape=[jax.ShapeDtypeStruct((HEADS, ROWS, CH), jnp.float32)] * 3,
    )(ea_t, wt, b, awt)


def _k1_body(hh_ref, pb_ref, w_ref, as_ref, ad_ref, xhh_ref, s_ref, d_ref):
    h = jnp.concatenate([hh_ref[k] for k in range(HEADS)], axis=1)
    h = jnp.maximum(h + pb_ref[...], 0.0)
    xh = h @ w_ref[...]
    for k in range(HEADS):
        xhh_ref[k] = xh[:, 16 * k:16 * k + 16]
    s_ref[...] = (h @ as_ref[...]).T
    d_ref[...] = (h @ ad_ref[...]).T


def _layer_proj(hh, pb, w, a_s, a_d):
    return pl.pallas_call(
        _k1_body,
        grid=(N_PAD // RB,),
        in_specs=[
            pl.BlockSpec((HEADS, RB, 16), lambda i: (0, i, 0)),
            pl.BlockSpec((1, HID), lambda i: (0, 0)),
            pl.BlockSpec((HID, HID), lambda i: (0, 0)),
            pl.BlockSpec((HID, HEADS), lambda i: (0, 0)),
            pl.BlockSpec((HID, HEADS), lambda i: (0, 0)),
        ],
        out_specs=[
            pl.BlockSpec((HEADS, RB, 16), lambda i: (0, i, 0)),
            pl.BlockSpec((HEADS, RB), lambda i: (0, i)),
            pl.BlockSpec((HEADS, RB), lambda i: (0, i)),
        ],
        out_shape=[
            jax.ShapeDtypeStruct((HEADS, N_PAD, 16), jnp.float32),
            jax.ShapeDtypeStruct((HEADS, N_PAD), jnp.float32),
            jax.ShapeDtypeStruct((HEADS, N_PAD), jnp.float32),
        ],
    )(hh, pb, w, a_s, a_d)


def _k3_body(dp_ref, o_ref):
    o_ref[...] = dp_ref[0] + dp_ref[1]


def _den_merge(denp):
    return pl.pallas_call(
        _k3_body,
        grid=(N_PAD // RB,),
        in_specs=[pl.BlockSpec((NC, HEADS, RB), lambda i: (0, 0, i))],
        out_specs=pl.BlockSpec((HEADS, RB), lambda i: (0, i)),
        out_shape=jax.ShapeDtypeStruct((HEADS, N_PAD), jnp.float32),
    )(denp)


def _kf_body(h_ref, b_ref, mw_ref, mb_ref, lw_ref, lb_ref, mu_ref, lv_ref):
    rid = lax.broadcasted_iota(jnp.int32, (N_PAD, 1), 0)
    h = jnp.maximum(h_ref[...] + b_ref[...], 0.0)
    h = jnp.where(rid < N, h, 0.0)
    hm = jnp.sum(h, axis=0, keepdims=True) * (1.0 / N)
    mu_ref[...] = hm @ mw_ref[...] + mb_ref[...]
    lv_ref[...] = hm @ lw_ref[...] + lb_ref[...]


def _final(hh, b, mw, mb, lw, lb):
    return pl.pallas_call(
        _kf_body,
        out_shape=(
            jax.ShapeDtypeStruct((1, LAT), jnp.float32),
            jax.ShapeDtypeStruct((1, LAT), jnp.float32),
        ),
    )(hh, b, mw, mb, lw, lb)


_MESH = plsc.VectorSubcoreMesh(core_axis_name="c", subcore_axis_name="s")
_SC_PARAMS = pltpu.CompilerParams(use_tc_tiling_on_sc=False)


@functools.partial(
    pl.kernel,
    out_type=(
        jax.ShapeDtypeStruct((ROWS, HEADS, CH), jnp.float32),
        jax.ShapeDtypeStruct((NC, HEADS, N_PAD), jnp.float32),
    ),
    mesh=_MESH,
    compiler_params=_SC_PARAMS,
    scratch_types=[
        pltpu.VMEM((1, CH), jnp.int32),
        pltpu.VMEM((1, CH), jnp.int32),
        pltpu.VMEM((HEADS, CH), jnp.float32),
        pltpu.VMEM((CH,), jnp.float32),
        pltpu.VMEM((CH,), jnp.float32),
        pltpu.VMEM((HEADS, CH), jnp.float32),
        pltpu.VMEM_SHARED((HEADS, N_PAD), jnp.float32),
        pltpu.VMEM_SHARED((HEADS, N_PAD), jnp.float32),
        pltpu.VMEM_SHARED((HEADS, N_PAD), jnp.float32),
    ],
)
def _sc_pass_a(src_hbm, dst_hbm, ae_hbm, asrc_hbm, adst_hbm, z4_hbm,
               ex_hbm, denp_hbm,
               idx_s, idx_d, ae_v, gs_v, gd_v, ex_v,
               asrc_sh, adst_sh, den_sh):
    c = lax.axis_index("c")
    s = lax.axis_index("s")
    w = s * NC + c
    row0 = s * NPT
    for h in range(HEADS):
        pltpu.sync_copy(asrc_hbm.at[h, pl.ds(row0, NPT)],
                        asrc_sh.at[h, pl.ds(row0, NPT)])
        pltpu.sync_copy(adst_hbm.at[h, pl.ds(row0, NPT)],
                        adst_sh.at[h, pl.ds(row0, NPT)])
        pltpu.sync_copy(z4_hbm.at[h, pl.ds(row0, NPT)],
                        den_sh.at[h, pl.ds(row0, NPT)])
    plsc.subcore_barrier()

    @pl.loop(0, ROWS_A)
    def _(j):
        r = w * ROWS_A + j
        pltpu.sync_copy(src_hbm.at[r], idx_s.at[0])
        pltpu.sync_copy(dst_hbm.at[r], idx_d.at[0])
        for h in range(HEADS):
            pltpu.sync_copy(ae_hbm.at[h, r], ae_v.at[h])
        for h in range(HEADS):
            pltpu.sync_copy(asrc_sh.at[h].at[idx_s.at[0]], gs_v)
            pltpu.sync_copy(adst_sh.at[h].at[idx_d.at[0]], gd_v)
            for i in range(CH // 16):
                a = (gs_v[pl.ds(16 * i, 16)] + gd_v[pl.ds(16 * i, 16)]
                     + ae_v[h, pl.ds(16 * i, 16)])
                a = jnp.where(a >= 0.0, a, 0.2 * a)
                ex_v[h, pl.ds(16 * i, 16)] = jnp.exp(a)
            pltpu.sync_copy(ex_v.at[h], den_sh.at[h].at[idx_d.at[0]], add=True)
        pltpu.sync_copy(ex_v, ex_hbm.at[r])

    plsc.subcore_barrier()
    for h in range(HEADS):
        pltpu.sync_copy(den_sh.at[h, pl.ds(row0, NPT)],
                        denp_hbm.at[c, h, pl.ds(row0, NPT)])


@functools.partial(
    pl.kernel,
    out_type=jax.ShapeDtypeStruct((HEADS, N_PAD, 16), jnp.float32),
    mesh=_MESH,
    compiler_params=_SC_PARAMS,
    scratch_types=[
        pltpu.VMEM((1, CH), jnp.int32),
        pltpu.VMEM((1, CH), jnp.int32),
        pltpu.VMEM((CH,), jnp.float32),
        pltpu.VMEM((CH,), jnp.float32),
        pltpu.VMEM((CH, 16), jnp.float32),
        pltpu.VMEM_SHARED((N_PAD,), jnp.float32),
        pltpu.VMEM_SHARED((N_PAD, 16), jnp.float32),
    ],
)
def _sc_pass_b(src_hbm, dst_hbm, ex_hbm, den_hbm, xh_hbm, z16_hbm,
               out_hbm,
               idx_s, idx_d, ex_h, w_h, rows_v, den_sh, out_sh):
    c = lax.axis_index("c")
    s = lax.axis_index("s")
    row0 = s * NPT
    for half in range(2):
        hd = c * 2 + half
        pltpu.sync_copy(den_hbm.at[hd, pl.ds(row0, NPT)],
                        den_sh.at[pl.ds(row0, NPT)])
        pltpu.sync_copy(z16_hbm.at[pl.ds(row0, NPT)],
                        out_sh.at[pl.ds(row0, NPT)])
        plsc.subcore_barrier()

        @pl.loop(0, ROWS_B)
        def _(j):
            r = s * ROWS_B + j
            pltpu.sync_copy(src_hbm.at[r], idx_s.at[0])
            pltpu.sync_copy(dst_hbm.at[r], idx_d.at[0])
            pltpu.sync_copy(ex_hbm.at[r, hd], ex_h)
            pltpu.sync_copy(den_sh.at[idx_d.at[0]], w_h)
            pltpu.sync_copy(xh_hbm.at[hd].at[idx_s.at[0]], rows_v)
            for i in range(CH // 16):
                w_h[pl.ds(16 * i, 16)] = (ex_h[pl.ds(16 * i, 16)]
                                          / w_h[pl.ds(16 * i, 16)])
            for e in range(CH):
                wrow = w_h[pl.ds((e // 16) * 16, 16)]
                wsp = jnp.broadcast_to(
                    lax.slice(wrow, (e % 16,), (e % 16 + 1,)), (16,))
                rows_v[e] = rows_v[e] * wsp
            pltpu.sync_copy(rows_v, out_sh.at[idx_d.at[0]], add=True)

        plsc.subcore_barrier()
        pltpu.sync_copy(out_sh.at[pl.ds(row0, NPT)],
                        out_hbm.at[hd, pl.ds(row0, NPT)])
        plsc.subcore_barrier()


def kernel(x, face_types, edge_index, edge_attr, params):
    p = params
    f32 = jnp.float32
    src = edge_index[0].astype(jnp.int32)
    dst = edge_index[1].astype(jnp.int32)
    pad_e = E_PAD - E
    src_p = jnp.concatenate([src, jnp.full((pad_e,), N, jnp.int32)]).reshape(ROWS, CH)
    dst_p = jnp.concatenate([dst, jnp.full((pad_e,), N, jnp.int32)]).reshape(ROWS, CH)
    ea_t = jnp.concatenate(
        [edge_attr, jnp.zeros((pad_e, EDGE_F), f32)]).T
    x_p = jnp.concatenate([x, jnp.zeros((N_PAD - N, NODE_F), f32)], axis=0)
    ft_p = jnp.concatenate(
        [face_types.astype(jnp.int32), jnp.zeros((N_PAD - N,), jnp.int32)]
    ).reshape(N_PAD, 1)

    def fold(wm, att):
        return (wm.reshape(HID, HEADS, OUT_C) * att[None]).sum(-1)

    ae_w = jnp.concatenate(
        [fold(p["gat"][l]["W_e"], p["gat"][l]["att_e"]) for l in range(LAYERS)],
        axis=1)
    a_srcs = [fold(p["gat"][l]["W"], p["gat"][l]["att_src"]) for l in range(LAYERS)]
    a_dsts = [fold(p["gat"][l]["W"], p["gat"][l]["att_dst"]) for l in range(LAYERS)]

    z4 = jnp.zeros((HEADS, N_PAD), f32)
    z16 = jnp.zeros((N_PAD, 16), f32)

    hh = _node_init(x_p, ft_p, p["face_emb"], p["node_W"], p["node_b"][None])
    ae_list = _edge_logits(ea_t, p["edge_W"].T, p["edge_b"][:, None], ae_w.T)

    prev_b = jnp.zeros((1, HID), f32)
    for l in range(LAYERS):
        g = p["gat"][l]
        xhh, asrc_t, adst_t = _layer_proj(hh, prev_b, g["W"], a_srcs[l], a_dsts[l])
        ex, denp = _sc_pass_a(src_p, dst_p, ae_list[l], asrc_t, adst_t, z4)
        den = _den_merge(denp)
        hh = _sc_pass_b(src_p, dst_p, ex, den, xhh, z16)
        prev_b = g["b"][None]

    hcat = jnp.concatenate([hh[k] for k in range(HEADS)], axis=1)
    mu, lv = _final(hcat, prev_b, p["mu_W"], p["mu_b"][None],
                    p["lv_W"], p["lv_b"][None])
    return (mu, lv)


# speedup vs baseline: 41.2853x; 1.3857x over previous
import functools

import jax
import jax.numpy as jnp
from jax import lax
from jax.experimental import pallas as pl
from jax.experimental.pallas import tpu as pltpu
from jax.experimental.pallas import tpu_sc as plsc

N = 50000
E = 800000
NODE_F = 13
EDGE_F = 2
FACE_D = 8
HID = 64
HEADS = 4
OUT_C = HID // HEADS
LAYERS = 3
LAT = 32

NC = 2
NS = 16
CH = 128

ROWS = 6272
E_PAD = ROWS * CH
ROWS_A = ROWS // (NC * NS)
ROWS_B = ROWS // NS
N_PAD = 50176
NPT = N_PAD // NS

RB = 1792
RBE = ROWS // 16


def _k0_body(x_ref, ft_ref, emb_ref, w_ref, b_ref, out_ref):
    xb = x_ref[...]
    ft = ft_ref[...]
    emb = emb_ref[...]
    fe = jnp.where(ft == 0, emb[0][None, :],
                   jnp.where(ft == 1, emb[1][None, :], emb[2][None, :]))
    xc = jnp.concatenate([xb, fe], axis=1)
    h = jnp.maximum(xc @ w_ref[...] + b_ref[...], 0.0)
    for k in range(HEADS):
        out_ref[k] = h[:, 16 * k:16 * k + 16]


def _node_init(x_p, ft_p, emb, w, b):
    return pl.pallas_call(
        _k0_body,
        grid=(N_PAD // RB,),
        in_specs=[
            pl.BlockSpec((RB, NODE_F), lambda i: (i, 0)),
            pl.BlockSpec((RB, 1), lambda i: (i, 0)),
            pl.BlockSpec((3, FACE_D), lambda i: (0, 0)),
            pl.BlockSpec((NODE_F + FACE_D, HID), lambda i: (0, 0)),
            pl.BlockSpec((1, HID), lambda i: (0, 0)),
        ],
        out_specs=pl.BlockSpec((HEADS, RB, 16), lambda i: (0, i, 0)),
        out_shape=jax.ShapeDtypeStruct((HEADS, N_PAD, 16), jnp.float32),
    )(x_p, ft_p, emb, w, b)


def _ke_body(ea_ref, wt_ref, b_ref, awt_ref, o0_ref, o1_ref, o2_ref):
    eh = jnp.maximum(wt_ref[...] @ ea_ref[...] + b_ref[...], 0.0)
    a = awt_ref[...] @ eh
    for l, o in enumerate((o0_ref, o1_ref, o2_ref)):
        o[...] = a[l * HEADS:(l + 1) * HEADS].reshape(HEADS, RBE, CH)


def _edge_logits(ea_t, wt, b, awt):
    return pl.pallas_call(
        _ke_body,
        grid=(ROWS // RBE,),
        in_specs=[
            pl.BlockSpec((EDGE_F, RBE * CH), lambda i: (0, i)),
            pl.BlockSpec((HID, EDGE_F), lambda i: (0, 0)),
            pl.BlockSpec((HID, 1), lambda i: (0, 0)),
            pl.BlockSpec((LAYERS * HEADS, HID), lambda i: (0, 0)),
        ],
        out_specs=[pl.BlockSpec((HEADS, RBE, CH), lambda i: (0, i, 0))] * 3,
        out_shape=[jax.ShapeDtypeStruct((HEADS, ROWS, CH), jnp.float32)] * 3,
    )(ea_t, wt, b, awt)


def _k1_body(hh_ref, pb_ref, w_ref, as_ref, ad_ref, xhh_ref, s_ref, d_ref):
    h = jnp.concatenate([hh_ref[k] for k in range(HEADS)], axis=1)
    h = jnp.maximum(h + pb_ref[...], 0.0)
    xh = h @ w_ref[...]
    for k in range(HEADS):
        xhh_ref[k] = xh[:, 16 * k:16 * k + 16]
    s_ref[...] = (h @ as_ref[...]).T
    d_ref[...] = (h @ ad_ref[...]).T


def _layer_proj(hh, pb, w, a_s, a_d):
    return pl.pallas_call(
        _k1_body,
        grid=(N_PAD // RB,),
        in_specs=[
            pl.BlockSpec((HEADS, RB, 16), lambda i: (0, i, 0)),
            pl.BlockSpec((1, HID), lambda i: (0, 0)),
            pl.BlockSpec((HID, HID), lambda i: (0, 0)),
            pl.BlockSpec((HID, HEADS), lambda i: (0, 0)),
            pl.BlockSpec((HID, HEADS), lambda i: (0, 0)),
        ],
        out_specs=[
            pl.BlockSpec((HEADS, RB, 16), lambda i: (0, i, 0)),
            pl.BlockSpec((HEADS, RB), lambda i: (0, i)),
            pl.BlockSpec((HEADS, RB), lambda i: (0, i)),
        ],
        out_shape=[
            jax.ShapeDtypeStruct((HEADS, N_PAD, 16), jnp.float32),
            jax.ShapeDtypeStruct((HEADS, N_PAD), jnp.float32),
            jax.ShapeDtypeStruct((HEADS, N_PAD), jnp.float32),
        ],
    )(hh, pb, w, a_s, a_d)


def _k3_body(dp_ref, o_ref):
    o_ref[...] = dp_ref[0] + dp_ref[1]


def _den_merge(denp):
    return pl.pallas_call(
        _k3_body,
        grid=(N_PAD // RB,),
        in_specs=[pl.BlockSpec((NC, HEADS, RB), lambda i: (0, 0, i))],
        out_specs=pl.BlockSpec((HEADS, RB), lambda i: (0, i)),
        out_shape=jax.ShapeDtypeStruct((HEADS, N_PAD), jnp.float32),
    )(denp)


def _kf_body(h_ref, b_ref, mw_ref, mb_ref, lw_ref, lb_ref, mu_ref, lv_ref):
    rid = lax.broadcasted_iota(jnp.int32, (N_PAD, 1), 0)
    h = jnp.maximum(h_ref[...] + b_ref[...], 0.0)
    h = jnp.where(rid < N, h, 0.0)
    hm = jnp.sum(h, axis=0, keepdims=True) * (1.0 / N)
    mu_ref[...] = hm @ mw_ref[...] + mb_ref[...]
    lv_ref[...] = hm @ lw_ref[...] + lb_ref[...]


def _final(hh, b, mw, mb, lw, lb):
    return pl.pallas_call(
        _kf_body,
        out_shape=(
            jax.ShapeDtypeStruct((1, LAT), jnp.float32),
            jax.ShapeDtypeStruct((1, LAT), jnp.float32),
        ),
    )(hh, b, mw, mb, lw, lb)


_MESH = plsc.VectorSubcoreMesh(core_axis_name="c", subcore_axis_name="s")
_SC_PARAMS = pltpu.CompilerParams(use_tc_tiling_on_sc=False)


@functools.partial(
    pl.kernel,
    out_type=(
        jax.ShapeDtypeStruct((HEADS, ROWS, CH), jnp.float32),
        jax.ShapeDtypeStruct((NC, HEADS, N_PAD), jnp.float32),
    ),
    mesh=_MESH,
    compiler_params=_SC_PARAMS,
    scratch_types=[
        pltpu.VMEM((1, CH), jnp.int32),
        pltpu.VMEM((1, CH), jnp.int32),
        pltpu.VMEM((HEADS, CH), jnp.float32),
        pltpu.VMEM((CH,), jnp.float32),
        pltpu.VMEM((CH,), jnp.float32),
        pltpu.VMEM((HEADS, CH), jnp.float32),
        pltpu.VMEM_SHARED((HEADS, N_PAD), jnp.float32),
        pltpu.VMEM_SHARED((HEADS, N_PAD), jnp.float32),
        pltpu.VMEM_SHARED((HEADS, N_PAD), jnp.float32),
    ],
)
def _sc_pass_a(src_hbm, dst_hbm, ae_hbm, asrc_hbm, adst_hbm, z4_hbm,
               ex_hbm, denp_hbm,
               idx_s, idx_d, ae_v, gs_v, gd_v, ex_v,
               asrc_sh, adst_sh, den_sh):
    c = lax.axis_index("c")
    s = lax.axis_index("s")
    w = s * NC + c
    row0 = s * NPT
    for h in range(HEADS):
        pltpu.sync_copy(asrc_hbm.at[h, pl.ds(row0, NPT)],
                        asrc_sh.at[h, pl.ds(row0, NPT)])
        pltpu.sync_copy(adst_hbm.at[h, pl.ds(row0, NPT)],
                        adst_sh.at[h, pl.ds(row0, NPT)])
        pltpu.sync_copy(z4_hbm.at[h, pl.ds(row0, NPT)],
                        den_sh.at[h, pl.ds(row0, NPT)])
    plsc.subcore_barrier()

    @pl.loop(0, ROWS_A)
    def _(j):
        r = w * ROWS_A + j
        pltpu.sync_copy(src_hbm.at[r], idx_s.at[0])
        pltpu.sync_copy(dst_hbm.at[r], idx_d.at[0])
        for h in range(HEADS):
            pltpu.sync_copy(ae_hbm.at[h, r], ae_v.at[h])
        for h in range(HEADS):
            pltpu.sync_copy(asrc_sh.at[h].at[idx_s.at[0]], gs_v)
            pltpu.sync_copy(adst_sh.at[h].at[idx_d.at[0]], gd_v)
            for i in range(CH // 16):
                a = (gs_v[pl.ds(16 * i, 16)] + gd_v[pl.ds(16 * i, 16)]
                     + ae_v[h, pl.ds(16 * i, 16)])
                a = jnp.where(a >= 0.0, a, 0.2 * a)
                ex_v[h, pl.ds(16 * i, 16)] = jnp.exp(a)
            pltpu.sync_copy(ex_v.at[h], den_sh.at[h].at[idx_d.at[0]], add=True)
            pltpu.sync_copy(ex_v.at[h], ex_hbm.at[h, r])

    plsc.subcore_barrier()
    for h in range(HEADS):
        pltpu.sync_copy(den_sh.at[h, pl.ds(row0, NPT)],
                        denp_hbm.at[c, h, pl.ds(row0, NPT)])


SCK = 8
NSC = ROWS_B // SCK


@functools.partial(
    pl.kernel,
    out_type=jax.ShapeDtypeStruct((HEADS, N_PAD, 16), jnp.float32),
    mesh=_MESH,
    compiler_params=_SC_PARAMS,
    scratch_types=[
        pltpu.VMEM((SCK, CH), jnp.int32),
        pltpu.VMEM((SCK, CH), jnp.int32),
        pltpu.VMEM((SCK, CH), jnp.float32),
        pltpu.VMEM((SCK, CH), jnp.float32),
        pltpu.VMEM((SCK * CH, 16), jnp.float32),
        pltpu.VMEM_SHARED((N_PAD,), jnp.float32),
        pltpu.VMEM_SHARED((N_PAD, 16), jnp.float32),
        pltpu.SemaphoreType.DMA,
        pltpu.SemaphoreType.DMA,
        pltpu.SemaphoreType.DMA,
    ],
)
def _sc_pass_b(src_hbm, dst_hbm, ex_hbm, den_hbm, xh_hbm, z16_hbm,
               out_hbm,
               isb, idb, exb, dnb, rows, den_sh, out_sh,
               sem_g0, sem_g1, sem_sc):
    c = lax.axis_index("c")
    s = lax.axis_index("s")
    row0 = s * NPT
    gsems = (sem_g0, sem_g1)
    for half in range(2):
        hd = c * 2 + half
        pltpu.sync_copy(den_hbm.at[hd, pl.ds(row0, NPT)],
                        den_sh.at[pl.ds(row0, NPT)])
        pltpu.sync_copy(z16_hbm.at[pl.ds(row0, NPT)],
                        out_sh.at[pl.ds(row0, NPT)])
        plsc.subcore_barrier()

        @pl.loop(0, NSC)
        def _(j):
            r0 = s * ROWS_B + j * SCK
            pltpu.sync_copy(src_hbm.at[pl.ds(r0, SCK)], isb)
            pltpu.sync_copy(dst_hbm.at[pl.ds(r0, SCK)], idb)
            pltpu.sync_copy(ex_hbm.at[hd, pl.ds(r0, SCK)], exb)

            sdesc = {}
            for k in range(SCK):
                pltpu.sync_copy(den_sh.at[idb.at[k]], dnb.at[k])
                pltpu.sync_copy(xh_hbm.at[hd].at[isb.at[k]],
                                rows.at[pl.ds(k * CH, CH)])
                for i in range(CH // 16):
                    dnb[k, pl.ds(16 * i, 16)] = (exb[k, pl.ds(16 * i, 16)]
                                                 / dnb[k, pl.ds(16 * i, 16)])
                for e in range(CH):
                    wrow = dnb[k, pl.ds((e // 16) * 16, 16)]
                    wsp = jnp.broadcast_to(
                        lax.slice(wrow, (e % 16,), (e % 16 + 1,)), (16,))
                    rr = k * CH + e
                    rows[rr] = rows[rr] * wsp
                sdesc[k] = pltpu.async_copy(rows.at[pl.ds(k * CH, CH)],
                                            out_sh.at[idb.at[k]], sem_sc,
                                            add=True)
                if k >= 2:
                    sdesc.pop(k - 2).wait()
            for k in sorted(sdesc):
                sdesc.pop(k).wait()

        plsc.subcore_barrier()
        pltpu.sync_copy(out_sh.at[pl.ds(row0, NPT)],
                        out_hbm.at[hd, pl.ds(row0, NPT)])
        plsc.subcore_barrier()


def kernel(x, face_types, edge_index, edge_attr, params):
    p = params
    f32 = jnp.float32
    src = edge_index[0].astype(jnp.int32)
    dst = edge_index[1].astype(jnp.int32)
    pad_e = E_PAD - E
    src_p = jnp.concatenate([src, jnp.full((pad_e,), N, jnp.int32)]).reshape(ROWS, CH)
    dst_p = jnp.concatenate([dst, jnp.full((pad_e,), N, jnp.int32)]).reshape(ROWS, CH)
    ea_t = jnp.concatenate(
        [edge_attr, jnp.zeros((pad_e, EDGE_F), f32)]).T
    x_p = jnp.concatenate([x, jnp.zeros((N_PAD - N, NODE_F), f32)], axis=0)
    ft_p = jnp.concatenate(
        [face_types.astype(jnp.int32), jnp.zeros((N_PAD - N,), jnp.int32)]
    ).reshape(N_PAD, 1)

    def fold(wm, att):
        return (wm.reshape(HID, HEADS, OUT_C) * att[None]).sum(-1)

    ae_w = jnp.concatenate(
        [fold(p["gat"][l]["W_e"], p["gat"][l]["att_e"]) for l in range(LAYERS)],
        axis=1)
    a_srcs = [fold(p["gat"][l]["W"], p["gat"][l]["att_src"]) for l in range(LAYERS)]
    a_dsts = [fold(p["gat"][l]["W"], p["gat"][l]["att_dst"]) for l in range(LAYERS)]

    z4 = jnp.zeros((HEADS, N_PAD), f32)
    z16 = jnp.zeros((N_PAD, 16), f32)

    hh = _node_init(x_p, ft_p, p["face_emb"], p["node_W"], p["node_b"][None])
    ae_list = _edge_logits(ea_t, p["edge_W"].T, p["edge_b"][:, None], ae_w.T)

    prev_b = jnp.zeros((1, HID), f32)
    for l in range(LAYERS):
        g = p["gat"][l]
        xhh, asrc_t, adst_t = _layer_proj(hh, prev_b, g["W"], a_srcs[l], a_dsts[l])
        ex, denp = _sc_pass_a(src_p, dst_p, ae_list[l], asrc_t, adst_t, z4)
        den = _den_merge(denp)
        hh = _sc_pass_b(src_p, dst_p, ex, den, xhh, z16)
        prev_b = g["b"][None]

    hcat = jnp.concatenate([hh[k] for k in range(HEADS)], axis=1)
    mu, lv = _final(hcat, prev_b, p["mu_W"], p["mu_b"][None],
                    p["lv_W"], p["lv_b"][None])
    return (mu, lv)


# speedup vs baseline: 54.0840x; 1.3100x over previous
import functools

import jax
import jax.numpy as jnp
from jax import lax
from jax.experimental import pallas as pl
from jax.experimental.pallas import tpu as pltpu
from jax.experimental.pallas import tpu_sc as plsc

N = 50000
E = 800000
NODE_F = 13
EDGE_F = 2
FACE_D = 8
HID = 64
HEADS = 4
OUT_C = HID // HEADS
LAYERS = 3
LAT = 32

NC = 2
NS = 16
CH = 128

ROWS = 6272
E_PAD = ROWS * CH
ROWS_A = ROWS // (NC * NS)
ROWS_B = ROWS // NS
N_PAD = 50176
NPT = N_PAD // NS

SCKA = 7

RB = 1792
RBE = ROWS // 16


def _k0_body(x_ref, ft_ref, emb_ref, w_ref, b_ref, out_ref):
    xb = x_ref[...]
    ft = ft_ref[...]
    emb = emb_ref[...]
    fe = jnp.where(ft == 0, emb[0][None, :],
                   jnp.where(ft == 1, emb[1][None, :], emb[2][None, :]))
    xc = jnp.concatenate([xb, fe], axis=1)
    h = jnp.maximum(xc @ w_ref[...] + b_ref[...], 0.0)
    for k in range(HEADS):
        out_ref[k] = h[:, 16 * k:16 * k + 16]


def _node_init(x_p, ft_p, emb, w, b):
    return pl.pallas_call(
        _k0_body,
        grid=(N_PAD // RB,),
        in_specs=[
            pl.BlockSpec((RB, NODE_F), lambda i: (i, 0)),
            pl.BlockSpec((RB, 1), lambda i: (i, 0)),
            pl.BlockSpec((3, FACE_D), lambda i: (0, 0)),
            pl.BlockSpec((NODE_F + FACE_D, HID), lambda i: (0, 0)),
            pl.BlockSpec((1, HID), lambda i: (0, 0)),
        ],
        out_specs=pl.BlockSpec((HEADS, RB, 16), lambda i: (0, i, 0)),
        out_shape=jax.ShapeDtypeStruct((HEADS, N_PAD, 16), jnp.float32),
    )(x_p, ft_p, emb, w, b)


def _ke_body(ea_ref, wt_ref, b_ref, awt_ref, o0_ref, o1_ref, o2_ref):
    eh = jnp.maximum(wt_ref[...] @ ea_ref[...] + b_ref[...], 0.0)
    a = awt_ref[...] @ eh
    for l, o in enumerate((o0_ref, o1_ref, o2_ref)):
        o[...] = a[l * HEADS:(l + 1) * HEADS].reshape(HEADS, RBE, CH)


def _edge_logits(ea_t, wt, b, awt):
    return pl.pallas_call(
        _ke_body,
        grid=(ROWS // RBE,),
        in_specs=[
            pl.BlockSpec((EDGE_F, RBE * CH), lambda i: (0, i)),
            pl.BlockSpec((HID, EDGE_F), lambda i: (0, 0)),
            pl.BlockSpec((HID, 1), lambda i: (0, 0)),
            pl.BlockSpec((LAYERS * HEADS, HID), lambda i: (0, 0)),
        ],
        out_specs=[pl.BlockSpec((HEADS, RBE, CH), lambda i: (0, i, 0))] * 3,
        out_shape=[jax.ShapeDtypeStruct((HEADS, ROWS, CH), jnp.float32)] * 3,
    )(ea_t, wt, b, awt)


def _k1_body(hh_ref, pb_ref, w_ref, as_ref, ad_ref, xhh_ref, s_ref, d_ref):
    h = jnp.concatenate([hh_ref[k] for k in range(HEADS)], axis=1)
    h = jnp.maximum(h + pb_ref[...], 0.0)
    xh = h @ w_ref[...]
    for k in range(HEADS):
        xhh_ref[k] = xh[:, 16 * k:16 * k + 16]
    s_ref[...] = (h @ as_ref[...]).T
    d_ref[...] = (h @ ad_ref[...]).T


def _layer_proj(hh, pb, w, a_s, a_d):
    return pl.pallas_call(
        _k1_body,
        grid=(N_PAD // RB,),
        in_specs=[
            pl.BlockSpec((HEADS, RB, 16), lambda i: (0, i, 0)),
            pl.BlockSpec((1, HID), lambda i: (0, 0)),
            pl.BlockSpec((HID, HID), lambda i: (0, 0)),
            pl.BlockSpec((HID, HEADS), lambda i: (0, 0)),
            pl.BlockSpec((HID, HEADS), lambda i: (0, 0)),
        ],
        out_specs=[
            pl.BlockSpec((HEADS, RB, 16), lambda i: (0, i, 0)),
            pl.BlockSpec((HEADS, RB), lambda i: (0, i)),
            pl.BlockSpec((HEADS, RB), lambda i: (0, i)),
        ],
        out_shape=[
            jax.ShapeDtypeStruct((HEADS, N_PAD, 16), jnp.float32),
            jax.ShapeDtypeStruct((HEADS, N_PAD), jnp.float32),
            jax.ShapeDtypeStruct((HEADS, N_PAD), jnp.float32),
        ],
    )(hh, pb, w, a_s, a_d)


def _k3_body(dp_ref, o_ref):
    o_ref[...] = dp_ref[0] + dp_ref[1]


def _den_merge(denp):
    return pl.pallas_call(
        _k3_body,
        grid=(N_PAD // RB,),
        in_specs=[pl.BlockSpec((NC, HEADS, RB), lambda i: (0, 0, i))],
        out_specs=pl.BlockSpec((HEADS, RB), lambda i: (0, i)),
        out_shape=jax.ShapeDtypeStruct((HEADS, N_PAD), jnp.float32),
    )(denp)


def _kf_body(h_ref, b_ref, mw_ref, mb_ref, lw_ref, lb_ref, mu_ref, lv_ref):
    rid = lax.broadcasted_iota(jnp.int32, (N_PAD, 1), 0)
    h = jnp.maximum(h_ref[...] + b_ref[...], 0.0)
    h = jnp.where(rid < N, h, 0.0)
    hm = jnp.sum(h, axis=0, keepdims=True) * (1.0 / N)
    mu_ref[...] = hm @ mw_ref[...] + mb_ref[...]
    lv_ref[...] = hm @ lw_ref[...] + lb_ref[...]


def _final(hh, b, mw, mb, lw, lb):
    return pl.pallas_call(
        _kf_body,
        out_shape=(
            jax.ShapeDtypeStruct((1, LAT), jnp.float32),
            jax.ShapeDtypeStruct((1, LAT), jnp.float32),
        ),
    )(hh, b, mw, mb, lw, lb)


_MESH = plsc.VectorSubcoreMesh(core_axis_name="c", subcore_axis_name="s")
_SC_PARAMS = pltpu.CompilerParams(use_tc_tiling_on_sc=False)


@functools.partial(
    pl.kernel,
    out_type=(
        jax.ShapeDtypeStruct((HEADS, ROWS, CH), jnp.float32),
        jax.ShapeDtypeStruct((NC, HEADS, N_PAD), jnp.float32),
    ),
    mesh=_MESH,
    compiler_params=_SC_PARAMS,
    scratch_types=[
        pltpu.VMEM((SCKA, CH), jnp.int32),
        pltpu.VMEM((SCKA, CH), jnp.int32),
        [pltpu.VMEM((SCKA, CH), jnp.float32)] * HEADS,
        [pltpu.VMEM((SCKA, CH), jnp.float32)] * HEADS,
        pltpu.VMEM((CH,), jnp.float32),
        pltpu.VMEM((CH,), jnp.float32),
        pltpu.VMEM_SHARED((HEADS, N_PAD), jnp.float32),
        pltpu.VMEM_SHARED((HEADS, N_PAD), jnp.float32),
        pltpu.VMEM_SHARED((HEADS, N_PAD), jnp.float32),
        pltpu.SemaphoreType.DMA,
        pltpu.SemaphoreType.DMA,
    ],
)
def _sc_pass_a(src_hbm, dst_hbm, ae_hbm, asrc_hbm, adst_hbm, z4_hbm,
               ex_hbm, denp_hbm,
               isb, idb, aeb, exb, gs_v, gd_v,
               asrc_sh, adst_sh, den_sh, sem_sc, sem_ex):
    c = lax.axis_index("c")
    s = lax.axis_index("s")
    w = s * NC + c
    row0 = s * NPT
    for h in range(HEADS):
        pltpu.sync_copy(asrc_hbm.at[h, pl.ds(row0, NPT)],
                        asrc_sh.at[h, pl.ds(row0, NPT)])
        pltpu.sync_copy(adst_hbm.at[h, pl.ds(row0, NPT)],
                        adst_sh.at[h, pl.ds(row0, NPT)])
        pltpu.sync_copy(z4_hbm.at[h, pl.ds(row0, NPT)],
                        den_sh.at[h, pl.ds(row0, NPT)])
    plsc.subcore_barrier()

    nsa = ROWS_A // SCKA

    @pl.loop(0, nsa)
    def _(j):
        r0 = w * ROWS_A + j * SCKA
        pltpu.sync_copy(src_hbm.at[pl.ds(r0, SCKA)], isb)
        pltpu.sync_copy(dst_hbm.at[pl.ds(r0, SCKA)], idb)
        for h in range(HEADS):
            pltpu.sync_copy(ae_hbm.at[h, pl.ds(r0, SCKA)], aeb[h])
        sdesc = []
        for k in range(SCKA):
            for h in range(HEADS):
                pltpu.sync_copy(asrc_sh.at[h].at[isb.at[k]], gs_v)
                pltpu.sync_copy(adst_sh.at[h].at[idb.at[k]], gd_v)
                for i in range(CH // 16):
                    a = (gs_v[pl.ds(16 * i, 16)] + gd_v[pl.ds(16 * i, 16)]
                         + aeb[h][k, pl.ds(16 * i, 16)])
                    a = jnp.where(a >= 0.0, a, 0.2 * a)
                    exb[h][k, pl.ds(16 * i, 16)] = jnp.exp(a)
                sdesc.append(pltpu.async_copy(
                    exb[h].at[k], den_sh.at[h].at[idb.at[k]], sem_sc,
                    add=True))
                if len(sdesc) > 2:
                    sdesc.pop(0).wait()
        edesc = [pltpu.async_copy(exb[h], ex_hbm.at[h, pl.ds(r0, SCKA)], sem_ex)
                 for h in range(HEADS)]
        for d in sdesc:
            d.wait()
        for d in edesc:
            d.wait()

    plsc.subcore_barrier()
    for h in range(HEADS):
        pltpu.sync_copy(den_sh.at[h, pl.ds(row0, NPT)],
                        denp_hbm.at[c, h, pl.ds(row0, NPT)])


SCK = 8
NSC = ROWS_B // SCK
SCKA = 7


@functools.partial(
    pl.kernel,
    out_type=jax.ShapeDtypeStruct((HEADS, N_PAD, 16), jnp.float32),
    mesh=_MESH,
    compiler_params=_SC_PARAMS,
    scratch_types=[
        pltpu.VMEM((SCK, CH), jnp.int32),
        pltpu.VMEM((SCK, CH), jnp.int32),
        pltpu.VMEM((SCK, CH), jnp.float32),
        pltpu.VMEM((SCK, CH), jnp.float32),
        pltpu.VMEM((SCK * CH, 16), jnp.float32),
        pltpu.VMEM_SHARED((N_PAD,), jnp.float32),
        pltpu.VMEM_SHARED((N_PAD, 16), jnp.float32),
        pltpu.SemaphoreType.DMA,
        pltpu.SemaphoreType.DMA,
        pltpu.SemaphoreType.DMA,
    ],
)
def _sc_pass_b(src_hbm, dst_hbm, ex_hbm, den_hbm, xh_hbm, z16_hbm,
               out_hbm,
               isb, idb, exb, dnb, rows, den_sh, out_sh,
               sem_g0, sem_g1, sem_sc):
    c = lax.axis_index("c")
    s = lax.axis_index("s")
    row0 = s * NPT
    gsems = (sem_g0, sem_g1)
    for half in range(2):
        hd = c * 2 + half
        pltpu.sync_copy(den_hbm.at[hd, pl.ds(row0, NPT)],
                        den_sh.at[pl.ds(row0, NPT)])
        pltpu.sync_copy(z16_hbm.at[pl.ds(row0, NPT)],
                        out_sh.at[pl.ds(row0, NPT)])
        plsc.subcore_barrier()

        @pl.loop(0, NSC)
        def _(j):
            r0 = s * ROWS_B + j * SCK
            pltpu.sync_copy(src_hbm.at[pl.ds(r0, SCK)], isb)
            pltpu.sync_copy(dst_hbm.at[pl.ds(r0, SCK)], idb)
            pltpu.sync_copy(ex_hbm.at[hd, pl.ds(r0, SCK)], exb)

            sdesc = {}
            for k in range(SCK):
                pltpu.sync_copy(den_sh.at[idb.at[k]], dnb.at[k])
                pltpu.sync_copy(xh_hbm.at[hd].at[isb.at[k]],
                                rows.at[pl.ds(k * CH, CH)])
                for i in range(CH // 16):
                    dnb[k, pl.ds(16 * i, 16)] = (exb[k, pl.ds(16 * i, 16)]
                                                 / dnb[k, pl.ds(16 * i, 16)])
                for e in range(CH):
                    wrow = dnb[k, pl.ds((e // 16) * 16, 16)]
                    wsp = jnp.broadcast_to(
                        lax.slice(wrow, (e % 16,), (e % 16 + 1,)), (16,))
                    rr = k * CH + e
                    rows[rr] = rows[rr] * wsp
                sdesc[k] = pltpu.async_copy(rows.at[pl.ds(k * CH, CH)],
                                            out_sh.at[idb.at[k]], sem_sc,
                                            add=True)
                if k >= 2:
                    sdesc.pop(k - 2).wait()
            for k in sorted(sdesc):
                sdesc.pop(k).wait()

        plsc.subcore_barrier()
        pltpu.sync_copy(out_sh.at[pl.ds(row0, NPT)],
                        out_hbm.at[hd, pl.ds(row0, NPT)])
        plsc.subcore_barrier()


def kernel(x, face_types, edge_index, edge_attr, params):
    p = params
    f32 = jnp.float32
    src = edge_index[0].astype(jnp.int32)
    dst = edge_index[1].astype(jnp.int32)
    pad_e = E_PAD - E
    src_p = jnp.concatenate([src, jnp.full((pad_e,), N, jnp.int32)]).reshape(ROWS, CH)
    dst_p = jnp.concatenate([dst, jnp.full((pad_e,), N, jnp.int32)]).reshape(ROWS, CH)
    ea_t = jnp.concatenate(
        [edge_attr, jnp.zeros((pad_e, EDGE_F), f32)]).T
    x_p = jnp.concatenate([x, jnp.zeros((N_PAD - N, NODE_F), f32)], axis=0)
    ft_p = jnp.concatenate(
        [face_types.astype(jnp.int32), jnp.zeros((N_PAD - N,), jnp.int32)]
    ).reshape(N_PAD, 1)

    def fold(wm, att):
        return (wm.reshape(HID, HEADS, OUT_C) * att[None]).sum(-1)

    ae_w = jnp.concatenate(
        [fold(p["gat"][l]["W_e"], p["gat"][l]["att_e"]) for l in range(LAYERS)],
        axis=1)
    a_srcs = [fold(p["gat"][l]["W"], p["gat"][l]["att_src"]) for l in range(LAYERS)]
    a_dsts = [fold(p["gat"][l]["W"], p["gat"][l]["att_dst"]) for l in range(LAYERS)]

    z4 = jnp.zeros((HEADS, N_PAD), f32)
    z16 = jnp.zeros((N_PAD, 16), f32)

    hh = _node_init(x_p, ft_p, p["face_emb"], p["node_W"], p["node_b"][None])
    ae_list = _edge_logits(ea_t, p["edge_W"].T, p["edge_b"][:, None], ae_w.T)

    prev_b = jnp.zeros((1, HID), f32)
    for l in range(LAYERS):
        g = p["gat"][l]
        xhh, asrc_t, adst_t = _layer_proj(hh, prev_b, g["W"], a_srcs[l], a_dsts[l])
        ex, denp = _sc_pass_a(src_p, dst_p, ae_list[l], asrc_t, adst_t, z4)
        den = _den_merge(denp)
        hh = _sc_pass_b(src_p, dst_p, ex, den, xhh, z16)
        prev_b = g["b"][None]

    hcat = jnp.concatenate([hh[k] for k in range(HEADS)], axis=1)
    mu, lv = _final(hcat, prev_b, p["mu_W"], p["mu_b"][None],
                    p["lv_W"], p["lv_b"][None])
    return (mu, lv)
